# Initial kernel scaffold; baseline (speedup 1.0000x reference)
import jax
import jax.numpy as jnp
from jax.experimental import pallas as pl


def kernel(coords, spatial_table, temporal_table, W1, b1, g1, be1, W2, b2, g2, be2, W3, b3):
    raise NotImplementedError("write your pallas kernel here")


# speedup vs baseline: 14.3723x; 14.3723x over previous
import functools

import numpy as np
import jax
import jax.numpy as jnp
from jax import lax
from jax.experimental import pallas as pl
from jax.experimental.pallas import tpu as pltpu
from jax.experimental.pallas import tpu_sc as plsc

SL = 36
TL = 20
ST = 2 ** 19
TT = 2 ** 16
F = 2
B = 131072
BASE = 16.0
GROWTH = 1.3819

P1_I32 = np.int32(np.asarray(2654435761, np.uint32).view(np.int32))
P2_I32 = np.int32(805459861)

RES_NP = np.array([np.floor(BASE * GROWTH ** l) for l in range(SL)], np.float32)

NC = 2
NS = 16
NW = NC * NS
PW = B // NW
C = 256
NCHUNK = PW // C
NG = C // 16
NR = C * 8

_SECTIONS = (
    ((0, 1, 2), SL, ST - 1, 0),
    ((0, 1, 3), TL, TT - 1, 2 * SL),
    ((0, 2, 3), TL, TT - 1, 2 * SL + 2 * TL),
    ((1, 2, 3), TL, TT - 1, 2 * SL + 4 * TL),
)


def _encode_body(coords_hbm, st_hbm, tt_hbm, res_hbm, out_hbm,
                 coords_v, frac_v, idx_v, rows_v, feats_v, res_v, sem):
    wid = lax.axis_index("s") * NC + lax.axis_index("c")
    base_pt = wid * PW
    pltpu.sync_copy(res_hbm, res_v)

    def chunk_body(ci, _):
        col = base_pt + ci * C
        pltpu.sync_copy(coords_hbm.at[:, pl.ds(col, C)], coords_v)

        for sec, (dims, nlev, mask, feat_base) in enumerate(_SECTIONS):
            d0, d1, d2 = dims
            tab = st_hbm if sec == 0 else tt_hbm
            tsize = ST if sec == 0 else TT
            sec_lev0 = 0 if sec == 0 else (sec - 1) * TL

            def level_body(l, _, d0=d0, d1=d1, d2=d2, tab=tab, tsize=tsize,
                           sec_lev0=sec_lev0, mask=mask, feat_base=feat_base,
                           nlev=nlev):
                resv = res_v[l, :]
                tbase2 = (sec_lev0 + l) * (tsize * 2)
                frow = feat_base + 2 * l

                def g_idx(g, _):
                    s = g * 16
                    x = coords_v[d0, pl.ds(s, 16)] * resv
                    y = coords_v[d1, pl.ds(s, 16)] * resv
                    z = coords_v[d2, pl.ds(s, 16)] * resv
                    xi = x.astype(jnp.int32)
                    yi = y.astype(jnp.int32)
                    zi = z.astype(jnp.int32)
                    frac_v[0, pl.ds(s, 16)] = x - xi.astype(jnp.float32)
                    frac_v[1, pl.ds(s, 16)] = y - yi.astype(jnp.float32)
                    frac_v[2, pl.ds(s, 16)] = z - zi.astype(jnp.float32)
                    hx0 = xi
                    hx1 = xi + 1
                    hy0 = yi * P1_I32
                    hy1 = hy0 + P1_I32
                    hz0 = zi * P2_I32
                    hz1 = hz0 + P2_I32
                    for j in range(8):
                        h = ((hx1 if j & 1 else hx0)
                             ^ (hy1 if j & 2 else hy0)
                             ^ (hz1 if j & 4 else hz0))
                        t = ((h & mask) << 1) + tbase2
                        idx_v[pl.ds((2 * j) * C + s, 16)] = t
                        idx_v[pl.ds((2 * j + 1) * C + s, 16)] = t + 1
                    return 0

                lax.fori_loop(0, NG, g_idx, 0, unroll=2)
                pltpu.async_copy(tab.at[idx_v], rows_v, sem).wait()

                def g_interp(g, _):
                    s = g * 16
                    fx = frac_v[0, pl.ds(s, 16)]
                    fy = frac_v[1, pl.ds(s, 16)]
                    fz = frac_v[2, pl.ds(s, 16)]
                    for c in range(2):
                        d = [rows_v[pl.ds((2 * j + c) * C + s, 16)]
                             for j in range(8)]
                        e0 = d[0] + fx * (d[1] - d[0])
                        e1 = d[2] + fx * (d[3] - d[2])
                        e2 = d[4] + fx * (d[5] - d[4])
                        e3 = d[6] + fx * (d[7] - d[6])
                        q0 = e0 + fy * (e1 - e0)
                        q1 = e2 + fy * (e3 - e2)
                        feats_v[frow + c, pl.ds(s, 16)] = q0 + fz * (q1 - q0)
                    return 0

                lax.fori_loop(0, NG, g_interp, 0, unroll=2)
                return 0

            lax.fori_loop(0, nlev, level_body, 0)

        pltpu.sync_copy(feats_v, out_hbm.at[:, pl.ds(col, C)])
        return 0

    lax.fori_loop(0, NCHUNK, chunk_body, 0)


def _encode(coords_t, st_flat, tt_flat):
    res_arr = jnp.asarray(np.repeat(RES_NP[:, None], 16, axis=1))
    mesh = plsc.VectorSubcoreMesh(core_axis_name="c", subcore_axis_name="s",
                                  num_cores=NC, num_subcores=NS)
    fn = pl.kernel(
        _encode_body,
        out_type=jax.ShapeDtypeStruct((2 * (SL + 3 * TL), B), jnp.float32),
        mesh=mesh,
        scratch_types=[
            pltpu.VMEM((4, C), jnp.float32),
            pltpu.VMEM((3, C), jnp.float32),
            pltpu.VMEM((NR * F,), jnp.int32),
            pltpu.VMEM((NR * F,), jnp.float32),
            pltpu.VMEM((2 * (SL + 3 * TL), C), jnp.float32),
            pltpu.VMEM((SL, 16), jnp.float32),
            pltpu.SemaphoreType.DMA,
        ],
    )
    return fn(coords_t, st_flat, tt_flat, res_arr)


def _mlp_body(x_ref, w1_ref, b1_ref, g1_ref, be1_ref,
              w2_ref, b2_ref, g2_ref, be2_ref, w3_ref, b3_ref, o_ref):
    x = x_ref[...]
    h = lax.dot_general(w1_ref[...], x, (((0,), (0,)), ((), ())),
                        preferred_element_type=jnp.float32,
                        precision=lax.Precision.HIGHEST)
    h = jnp.maximum(h + b1_ref[...], 0.0)
    m = jnp.mean(h, axis=0, keepdims=True)
    v = jnp.mean((h - m) ** 2, axis=0, keepdims=True)
    h = (h - m) * lax.rsqrt(v + 1e-5) * g1_ref[...] + be1_ref[...]
    h = lax.dot_general(w2_ref[...], h, (((0,), (0,)), ((), ())),
                        preferred_element_type=jnp.float32,
                        precision=lax.Precision.HIGHEST)
    h = jnp.maximum(h + b2_ref[...], 0.0)
    m = jnp.mean(h, axis=0, keepdims=True)
    v = jnp.mean((h - m) ** 2, axis=0, keepdims=True)
    h = (h - m) * lax.rsqrt(v + 1e-5) * g2_ref[...] + be2_ref[...]
    o = lax.dot_general(w3_ref[...], h, (((0,), (0,)), ((), ())),
                        preferred_element_type=jnp.float32,
                        precision=lax.Precision.HIGHEST)
    o_ref[...] = o + b3_ref[...]


def _mlp(feats, W1, b1, g1, be1, W2, b2, g2, be2, W3, b3):
    D = feats.shape[0]
    H = W1.shape[1]
    BT = 2048
    grid = (B // BT,)
    full = lambda i: (0, 0)
    out = pl.pallas_call(
        _mlp_body,
        grid=grid,
        in_specs=[
            pl.BlockSpec((D, BT), lambda i: (0, i)),
            pl.BlockSpec((D, H), full),
            pl.BlockSpec((H, 1), full),
            pl.BlockSpec((H, 1), full),
            pl.BlockSpec((H, 1), full),
            pl.BlockSpec((H, H), full),
            pl.BlockSpec((H, 1), full),
            pl.BlockSpec((H, 1), full),
            pl.BlockSpec((H, 1), full),
            pl.BlockSpec((H, 1), full),
            pl.BlockSpec((1, 1), full),
        ],
        out_specs=pl.BlockSpec((1, BT), lambda i: (0, i)),
        out_shape=jax.ShapeDtypeStruct((1, B), jnp.float32),
    )(feats, W1, b1.reshape(H, 1), g1.reshape(H, 1), be1.reshape(H, 1),
      W2, b2.reshape(H, 1), g2.reshape(H, 1), be2.reshape(H, 1),
      W3, b3.reshape(1, 1))
    return out[0]


def kernel(coords, spatial_table, temporal_table,
           W1, b1, g1, be1, W2, b2, g2, be2, W3, b3):
    coords_t = coords.T
    st_flat = spatial_table.reshape(SL * ST * F)
    tt_flat = temporal_table.reshape(3 * TL * TT * F)
    feats = _encode(coords_t, st_flat, tt_flat)
    return _mlp(feats, W1, b1, g1, be1, W2, b2, g2, be2, W3, b3)


# speedup vs baseline: 54.5148x; 3.7930x over previous
import functools

import numpy as np
import jax
import jax.numpy as jnp
from jax import lax
from jax.experimental import pallas as pl
from jax.experimental.pallas import tpu as pltpu
from jax.experimental.pallas import tpu_sc as plsc

SL = 36
TL = 20
ST = 2 ** 19
TT = 2 ** 16
F = 2
B = 131072
BASE = 16.0
GROWTH = 1.3819

P1_I32 = np.int32(np.asarray(2654435761, np.uint32).view(np.int32))
P2_I32 = np.int32(805459861)

RES_NP = np.array([np.floor(BASE * GROWTH ** l) for l in range(SL)], np.float32)

NC = 2
NS = 16
NW = NC * NS
PW = B // NW
C = 256
NCHUNK = PW // C
NG = C // 16
NR = C * 8

_SECTIONS = (
    ((0, 1, 2), SL, ST - 1, 0),
    ((0, 1, 3), TL, TT - 1, 2 * SL),
    ((0, 2, 3), TL, TT - 1, 2 * SL + 2 * TL),
    ((1, 2, 3), TL, TT - 1, 2 * SL + 4 * TL),
)


def _encode_body(coords_hbm, st_hbm, tt_hbm, res_hbm, out_hbm,
                 coords_v, frac_v, idx_v, rows_v, feats_v, res_v, sem):
    wid = lax.axis_index("s") * NC + lax.axis_index("c")
    base_pt = wid * PW
    pltpu.sync_copy(res_hbm, res_v)

    def chunk_body(ci, _):
        col = base_pt + ci * C
        pltpu.sync_copy(coords_hbm.at[pl.ds(col * 4, C * 4)], coords_v)

        for sec, (dims, nlev, mask, feat_base) in enumerate(_SECTIONS):
            d0, d1, d2 = dims
            tab = st_hbm if sec == 0 else tt_hbm
            tsize = ST if sec == 0 else TT
            sec_lev0 = 0 if sec == 0 else (sec - 1) * TL

            def level_body(l, _, d0=d0, d1=d1, d2=d2, tab=tab, tsize=tsize,
                           sec_lev0=sec_lev0, mask=mask, feat_base=feat_base,
                           nlev=nlev):
                resv = res_v[l, :]
                tbase2 = (sec_lev0 + l) * (tsize * 2)
                frow = feat_base + 2 * l

                def g_idx(g, _):
                    s = g * 16
                    cb = (g >> 3) * 512 + (g & 7) * 16
                    x = coords_v[pl.ds(cb + d0 * 128, 16)] * resv
                    y = coords_v[pl.ds(cb + d1 * 128, 16)] * resv
                    z = coords_v[pl.ds(cb + d2 * 128, 16)] * resv
                    xi = x.astype(jnp.int32)
                    yi = y.astype(jnp.int32)
                    zi = z.astype(jnp.int32)
                    frac_v[0, pl.ds(s, 16)] = x - xi.astype(jnp.float32)
                    frac_v[1, pl.ds(s, 16)] = y - yi.astype(jnp.float32)
                    frac_v[2, pl.ds(s, 16)] = z - zi.astype(jnp.float32)
                    hx0 = xi
                    hx1 = xi + 1
                    hy0 = yi * P1_I32
                    hy1 = hy0 + P1_I32
                    hz0 = zi * P2_I32
                    hz1 = hz0 + P2_I32
                    for j in range(8):
                        h = ((hx1 if j & 1 else hx0)
                             ^ (hy1 if j & 2 else hy0)
                             ^ (hz1 if j & 4 else hz0))
                        r = h & mask
                        t = ((r >> 7) << 8) + (r & 127) + tbase2
                        idx_v[pl.ds((2 * j) * C + s, 16)] = t
                        idx_v[pl.ds((2 * j + 1) * C + s, 16)] = t + 128
                    return 0

                lax.fori_loop(0, NG, g_idx, 0, unroll=2)
                pltpu.async_copy(tab.at[idx_v], rows_v, sem).wait()

                def g_interp(g, _):
                    s = g * 16
                    fx = frac_v[0, pl.ds(s, 16)]
                    fy = frac_v[1, pl.ds(s, 16)]
                    fz = frac_v[2, pl.ds(s, 16)]
                    for c in range(2):
                        d = [rows_v[pl.ds((2 * j + c) * C + s, 16)]
                             for j in range(8)]
                        e0 = d[0] + fx * (d[1] - d[0])
                        e1 = d[2] + fx * (d[3] - d[2])
                        e2 = d[4] + fx * (d[5] - d[4])
                        e3 = d[6] + fx * (d[7] - d[6])
                        q0 = e0 + fy * (e1 - e0)
                        q1 = e2 + fy * (e3 - e2)
                        feats_v[frow + c, pl.ds(s, 16)] = q0 + fz * (q1 - q0)
                    return 0

                lax.fori_loop(0, NG, g_interp, 0, unroll=2)
                return 0

            lax.fori_loop(0, nlev, level_body, 0)

        pltpu.sync_copy(feats_v, out_hbm.at[:, pl.ds(col, C)])
        return 0

    lax.fori_loop(0, NCHUNK, chunk_body, 0)


def _encode(coords_t, st_flat, tt_flat):
    res_arr = jnp.asarray(np.repeat(RES_NP[:, None], 16, axis=1))
    mesh = plsc.VectorSubcoreMesh(core_axis_name="c", subcore_axis_name="s",
                                  num_cores=NC, num_subcores=NS)
    fn = pl.kernel(
        _encode_body,
        out_type=jax.ShapeDtypeStruct((2 * (SL + 3 * TL), B), jnp.float32),
        mesh=mesh,
        scratch_types=[
            pltpu.VMEM((4 * C,), jnp.float32),
            pltpu.VMEM((3, C), jnp.float32),
            pltpu.VMEM((NR * F,), jnp.int32),
            pltpu.VMEM((NR * F,), jnp.float32),
            pltpu.VMEM((2 * (SL + 3 * TL), C), jnp.float32),
            pltpu.VMEM((SL, 16), jnp.float32),
            pltpu.SemaphoreType.DMA,
        ],
    )
    return fn(coords_t, st_flat, tt_flat, res_arr)


def _mlp_body(x_ref, w1_ref, b1_ref, g1_ref, be1_ref,
              w2_ref, b2_ref, g2_ref, be2_ref, w3_ref, b3_ref, o_ref):
    x = x_ref[...]
    h = lax.dot_general(w1_ref[...], x, (((0,), (0,)), ((), ())),
                        preferred_element_type=jnp.float32,
                        precision=lax.Precision.HIGHEST)
    h = jnp.maximum(h + b1_ref[...], 0.0)
    m = jnp.mean(h, axis=0, keepdims=True)
    v = jnp.mean((h - m) ** 2, axis=0, keepdims=True)
    h = (h - m) * lax.rsqrt(v + 1e-5) * g1_ref[...] + be1_ref[...]
    h = lax.dot_general(w2_ref[...], h, (((0,), (0,)), ((), ())),
                        preferred_element_type=jnp.float32,
                        precision=lax.Precision.HIGHEST)
    h = jnp.maximum(h + b2_ref[...], 0.0)
    m = jnp.mean(h, axis=0, keepdims=True)
    v = jnp.mean((h - m) ** 2, axis=0, keepdims=True)
    h = (h - m) * lax.rsqrt(v + 1e-5) * g2_ref[...] + be2_ref[...]
    o = lax.dot_general(w3_ref[...], h, (((0,), (0,)), ((), ())),
                        preferred_element_type=jnp.float32,
                        precision=lax.Precision.HIGHEST)
    o_ref[...] = o + b3_ref[...]


def _mlp(feats, W1, b1, g1, be1, W2, b2, g2, be2, W3, b3):
    D = feats.shape[0]
    H = W1.shape[1]
    BT = 2048
    grid = (B // BT,)
    full = lambda i: (0, 0)
    out = pl.pallas_call(
        _mlp_body,
        grid=grid,
        in_specs=[
            pl.BlockSpec((D, BT), lambda i: (0, i)),
            pl.BlockSpec((D, H), full),
            pl.BlockSpec((H, 1), full),
            pl.BlockSpec((H, 1), full),
            pl.BlockSpec((H, 1), full),
            pl.BlockSpec((H, H), full),
            pl.BlockSpec((H, 1), full),
            pl.BlockSpec((H, 1), full),
            pl.BlockSpec((H, 1), full),
            pl.BlockSpec((H, 1), full),
            pl.BlockSpec((1, 1), full),
        ],
        out_specs=pl.BlockSpec((1, BT), lambda i: (0, i)),
        out_shape=jax.ShapeDtypeStruct((1, B), jnp.float32),
    )(feats, W1, b1.reshape(H, 1), g1.reshape(H, 1), be1.reshape(H, 1),
      W2, b2.reshape(H, 1), g2.reshape(H, 1), be2.reshape(H, 1),
      W3, b3.reshape(1, 1))
    return out[0]


def kernel(coords, spatial_table, temporal_table,
           W1, b1, g1, be1, W2, b2, g2, be2, W3, b3):
    st_flat = (spatial_table.reshape(SL, ST // 128, 128, F)
               .transpose(0, 1, 3, 2).reshape(SL * ST * F))
    tt_flat = (temporal_table.reshape(3, TL, TT // 128, 128, F)
               .transpose(0, 1, 2, 4, 3).reshape(3 * TL * TT * F))
    coords_f = (coords.reshape(B // 128, 128, 4)
                .transpose(0, 2, 1).reshape(B * 4))
    feats = _encode(coords_f, st_flat, tt_flat)
    return _mlp(feats, W1, b1, g1, be1, W2, b2, g2, be2, W3, b3)


# speedup vs baseline: 73.8364x; 1.3544x over previous
import functools

import numpy as np
import jax
import jax.numpy as jnp
from jax import lax
from jax.experimental import pallas as pl
from jax.experimental.pallas import tpu as pltpu
from jax.experimental.pallas import tpu_sc as plsc

SL = 36
TL = 20
ST = 2 ** 19
TT = 2 ** 16
F = 2
B = 131072
BASE = 16.0
GROWTH = 1.3819

P1_I32 = np.int32(np.asarray(2654435761, np.uint32).view(np.int32))
P2_I32 = np.int32(805459861)

RES_NP = np.array([np.floor(BASE * GROWTH ** l) for l in range(SL)], np.float32)

NC = 2
NS = 16
NW = NC * NS
PW = B // NW
C = 256
NCHUNK = PW // C
NG = C // 16
NR = C * 8

_SECTIONS = (
    ((0, 1, 2), SL, ST - 1, 0),
    ((0, 1, 3), TL, TT - 1, 2 * SL),
    ((0, 2, 3), TL, TT - 1, 2 * SL + 2 * TL),
    ((1, 2, 3), TL, TT - 1, 2 * SL + 4 * TL),
)


NRW = NR * F


def _encode_body(coords_hbm, st_hbm, tt_hbm, res_hbm, out_hbm,
                 coords_v, frac_a, frac_b, idx_a, idx_b, rows_a, rows_b,
                 feats_v, res_v, sem_a, sem_b):
    wid = lax.axis_index("s") * NC + lax.axis_index("c")
    base_pt = wid * PW
    pltpu.sync_copy(res_hbm, res_v)
    sems = (sem_a, sem_b)
    fracs = (frac_a, frac_b)
    idxs = (idx_a, idx_b)
    rows = (rows_a, rows_b)

    def build(resrow, d0o, d1o, d2o, mask, tbase2, slot):
        resv = res_v[resrow, :]
        frac_v = fracs[slot]
        idx_v = idxs[slot]

        def g_idx(g, _):
            s = g * 16
            cb = (g >> 3) * 512 + (g & 7) * 16
            x = coords_v[pl.ds(cb + d0o, 16)] * resv
            y = coords_v[pl.ds(cb + d1o, 16)] * resv
            z = coords_v[pl.ds(cb + d2o, 16)] * resv
            xi = x.astype(jnp.int32)
            yi = y.astype(jnp.int32)
            zi = z.astype(jnp.int32)
            frac_v[0, pl.ds(s, 16)] = x - xi.astype(jnp.float32)
            frac_v[1, pl.ds(s, 16)] = y - yi.astype(jnp.float32)
            frac_v[2, pl.ds(s, 16)] = z - zi.astype(jnp.float32)
            hx0 = xi
            hx1 = xi + 1
            hy0 = yi * P1_I32
            hy1 = hy0 + P1_I32
            hz0 = zi * P2_I32
            hz1 = hz0 + P2_I32
            for j in range(8):
                h = ((hx1 if j & 1 else hx0)
                     ^ (hy1 if j & 2 else hy0)
                     ^ (hz1 if j & 4 else hz0))
                r = h & mask
                t = (r + (r & -128)) + tbase2
                idx_v[pl.ds((2 * j) * C + s, 16)] = t
                idx_v[pl.ds((2 * j + 1) * C + s, 16)] = t + 128
            return 0

        lax.fori_loop(0, NG, g_idx, 0, unroll=2)

    def fire(tab, slot):
        pltpu.async_copy(tab.at[idxs[slot]], rows[slot], sems[slot])

    def drain(tab, slot):
        pltpu.make_async_copy(tab.at[idxs[slot]], rows[slot],
                              sems[slot]).wait()

    def interp(frow, slot):
        frac_v = fracs[slot]
        rows_v = rows[slot]

        def g_interp(g, _):
            s = g * 16
            fx = frac_v[0, pl.ds(s, 16)]
            fy = frac_v[1, pl.ds(s, 16)]
            fz = frac_v[2, pl.ds(s, 16)]
            for c in range(2):
                d = [rows_v[pl.ds((2 * j + c) * C + s, 16)]
                     for j in range(8)]
                e0 = d[0] + fx * (d[1] - d[0])
                e1 = d[2] + fx * (d[3] - d[2])
                e2 = d[4] + fx * (d[5] - d[4])
                e3 = d[6] + fx * (d[7] - d[6])
                q0 = e0 + fy * (e1 - e0)
                q1 = e2 + fy * (e3 - e2)
                feats_v[frow + c, pl.ds(s, 16)] = q0 + fz * (q1 - q0)
            return 0

        lax.fori_loop(0, NG, g_interp, 0, unroll=2)

    def run_section(nlev, tab, build_l, frow_l):
        build_l(0, 0)
        fire(tab, 0)

        def body(k, _):
            l0 = 2 * k
            build_l(l0 + 1, 1)
            fire(tab, 1)
            drain(tab, 0)
            interp(frow_l(l0), 0)
            build_l(l0 + 2, 0)
            fire(tab, 0)
            drain(tab, 1)
            interp(frow_l(l0 + 1), 1)
            return 0

        lax.fori_loop(0, nlev // 2 - 1, body, 0)
        build_l(nlev - 1, 1)
        fire(tab, 1)
        drain(tab, 0)
        interp(frow_l(nlev - 2), 0)
        drain(tab, 1)
        interp(frow_l(nlev - 1), 1)

    def chunk_body(ci, _):
        col = base_pt + ci * C
        pltpu.sync_copy(coords_hbm.at[pl.ds(col * 4, C * 4)], coords_v)

        def build_sp(l, slot):
            build(l, 0, 128, 256, ST - 1, l * (ST * 2), slot)

        run_section(SL, st_hbm, build_sp, lambda l: 2 * l)

        def build_tm(u, slot):
            i = u // TL
            d0o = jnp.where(i >= 2, 128, 0)
            d1o = jnp.where(i >= 1, 256, 128)
            build(SL + u, d0o, d1o, 384, TT - 1, u * (TT * 2), slot)

        run_section(3 * TL, tt_hbm, build_tm, lambda u: 2 * SL + 2 * u)

        pltpu.sync_copy(feats_v, out_hbm.at[:, pl.ds(col, C)])
        return 0

    lax.fori_loop(0, NCHUNK, chunk_body, 0)


def _encode(coords_t, st_flat, tt_flat):
    res96 = np.concatenate([RES_NP, np.tile(RES_NP[:TL], 3)])
    res_arr = jnp.asarray(np.repeat(res96[:, None], 16, axis=1))
    mesh = plsc.VectorSubcoreMesh(core_axis_name="c", subcore_axis_name="s",
                                  num_cores=NC, num_subcores=NS)
    fn = pl.kernel(
        _encode_body,
        out_type=jax.ShapeDtypeStruct((2 * (SL + 3 * TL), B), jnp.float32),
        mesh=mesh,
        scratch_types=[
            pltpu.VMEM((4 * C,), jnp.float32),
            pltpu.VMEM((3, C), jnp.float32),
            pltpu.VMEM((3, C), jnp.float32),
            pltpu.VMEM((NRW,), jnp.int32),
            pltpu.VMEM((NRW,), jnp.int32),
            pltpu.VMEM((NRW,), jnp.float32),
            pltpu.VMEM((NRW,), jnp.float32),
            pltpu.VMEM((2 * (SL + 3 * TL), C), jnp.float32),
            pltpu.VMEM((SL + 3 * TL, 16), jnp.float32),
            pltpu.SemaphoreType.DMA,
            pltpu.SemaphoreType.DMA,
        ],
    )
    return fn(coords_t, st_flat, tt_flat, res_arr)


def _mlp_body(x_ref, w1_ref, b1_ref, g1_ref, be1_ref,
              w2_ref, b2_ref, g2_ref, be2_ref, w3_ref, b3_ref, o_ref):
    x = x_ref[...]
    h = lax.dot_general(w1_ref[...], x, (((0,), (0,)), ((), ())),
                        preferred_element_type=jnp.float32,
                        precision=lax.Precision.HIGHEST)
    h = jnp.maximum(h + b1_ref[...], 0.0)
    m = jnp.mean(h, axis=0, keepdims=True)
    v = jnp.mean((h - m) ** 2, axis=0, keepdims=True)
    h = (h - m) * lax.rsqrt(v + 1e-5) * g1_ref[...] + be1_ref[...]
    h = lax.dot_general(w2_ref[...], h, (((0,), (0,)), ((), ())),
                        preferred_element_type=jnp.float32,
                        precision=lax.Precision.HIGHEST)
    h = jnp.maximum(h + b2_ref[...], 0.0)
    m = jnp.mean(h, axis=0, keepdims=True)
    v = jnp.mean((h - m) ** 2, axis=0, keepdims=True)
    h = (h - m) * lax.rsqrt(v + 1e-5) * g2_ref[...] + be2_ref[...]
    o = lax.dot_general(w3_ref[...], h, (((0,), (0,)), ((), ())),
                        preferred_element_type=jnp.float32,
                        precision=lax.Precision.HIGHEST)
    o_ref[...] = o + b3_ref[...]


def _mlp(feats, W1, b1, g1, be1, W2, b2, g2, be2, W3, b3):
    D = feats.shape[0]
    H = W1.shape[1]
    BT = 2048
    grid = (B // BT,)
    full = lambda i: (0, 0)
    out = pl.pallas_call(
        _mlp_body,
        grid=grid,
        in_specs=[
            pl.BlockSpec((D, BT), lambda i: (0, i)),
            pl.BlockSpec((D, H), full),
            pl.BlockSpec((H, 1), full),
            pl.BlockSpec((H, 1), full),
            pl.BlockSpec((H, 1), full),
            pl.BlockSpec((H, H), full),
            pl.BlockSpec((H, 1), full),
            pl.BlockSpec((H, 1), full),
            pl.BlockSpec((H, 1), full),
            pl.BlockSpec((H, 1), full),
            pl.BlockSpec((1, 1), full),
        ],
        out_specs=pl.BlockSpec((1, BT), lambda i: (0, i)),
        out_shape=jax.ShapeDtypeStruct((1, B), jnp.float32),
    )(feats, W1, b1.reshape(H, 1), g1.reshape(H, 1), be1.reshape(H, 1),
      W2, b2.reshape(H, 1), g2.reshape(H, 1), be2.reshape(H, 1),
      W3, b3.reshape(1, 1))
    return out[0]


def kernel(coords, spatial_table, temporal_table,
           W1, b1, g1, be1, W2, b2, g2, be2, W3, b3):
    st_flat = (spatial_table.reshape(SL, ST // 128, 128, F)
               .transpose(0, 1, 3, 2).reshape(SL * ST * F))
    tt_flat = (temporal_table.reshape(3, TL, TT // 128, 128, F)
               .transpose(0, 1, 2, 4, 3).reshape(3 * TL * TT * F))
    coords_f = (coords.reshape(B // 128, 128, 4)
                .transpose(0, 2, 1).reshape(B * 4))
    feats = _encode(coords_f, st_flat, tt_flat)
    return _mlp(feats, W1, b1, g1, be1, W2, b2, g2, be2, W3, b3)


# speedup vs baseline: 129.0325x; 1.7475x over previous
import functools

import numpy as np
import jax
import jax.numpy as jnp
from jax import lax
from jax.experimental import pallas as pl
from jax.experimental.pallas import tpu as pltpu
from jax.experimental.pallas import tpu_sc as plsc

SL = 36
TL = 20
ST = 2 ** 19
TT = 2 ** 16
F = 2
B = 131072
BASE = 16.0
GROWTH = 1.3819

P1_I32 = np.int32(np.asarray(2654435761, np.uint32).view(np.int32))
P2_I32 = np.int32(805459861)

RES_NP = np.array([np.floor(BASE * GROWTH ** l) for l in range(SL)], np.float32)

NC = 2
NS = 16
NW = NC * NS
PW = B // NW
C = 256
NCHUNK = PW // C
NG = C // 16
NR = C * 8

_SECTIONS = (
    ((0, 1, 2), SL, ST - 1, 0),
    ((0, 1, 3), TL, TT - 1, 2 * SL),
    ((0, 2, 3), TL, TT - 1, 2 * SL + 2 * TL),
    ((1, 2, 3), TL, TT - 1, 2 * SL + 4 * TL),
)


NRW = NR * F
SLICE_W = (TT * F) // NS


def _encode_body(coords_hbm, st_hbm, tt_hbm, res_hbm, out_hbm,
                 coords_v, frac_a, frac_b, idx_a, idx_b, rows_a, rows_b,
                 feats_sp, flvl_a, flvl_b, res_v, spm_a, spm_b,
                 sem_a, sem_b, sem_sa, sem_sb, sem_wa, sem_wb):
    cid = lax.axis_index("c")
    sid = lax.axis_index("s")
    wid = sid * NC + cid
    base_pt = wid * PW
    pltpu.sync_copy(res_hbm, res_v)
    pltpu.sync_copy(coords_hbm.at[pl.ds(base_pt * 4, PW * 4)], coords_v)
    sems = (sem_a, sem_b)
    fracs = (frac_a, frac_b)
    idxs = (idx_a, idx_b)
    rows = (rows_a, rows_b)
    spms = (spm_a, spm_b)
    sem_s = (sem_sa, sem_sb)
    sem_w = (sem_wa, sem_wb)
    flvls = (flvl_a, flvl_b)

    def build(pb, resrow, d0o, d1o, d2o, mask, tbase2, slot):
        resv = res_v[resrow, :]
        frac_v = fracs[slot]
        idx_v = idxs[slot]

        def g_idx(g, _):
            s = g * 16
            cb = pb + (g >> 3) * 512 + (g & 7) * 16
            x = coords_v[pl.ds(cb + d0o, 16)] * resv
            y = coords_v[pl.ds(cb + d1o, 16)] * resv
            z = coords_v[pl.ds(cb + d2o, 16)] * resv
            xi = x.astype(jnp.int32)
            yi = y.astype(jnp.int32)
            zi = z.astype(jnp.int32)
            frac_v[0, pl.ds(s, 16)] = x - xi.astype(jnp.float32)
            frac_v[1, pl.ds(s, 16)] = y - yi.astype(jnp.float32)
            frac_v[2, pl.ds(s, 16)] = z - zi.astype(jnp.float32)
            hx0 = xi
            hx1 = xi + 1
            hy0 = yi * P1_I32
            hy1 = hy0 + P1_I32
            hz0 = zi * P2_I32
            hz1 = hz0 + P2_I32
            for j in range(8):
                h = ((hx1 if j & 1 else hx0)
                     ^ (hy1 if j & 2 else hy0)
                     ^ (hz1 if j & 4 else hz0))
                r = h & mask
                t = (r + (r & -128)) + tbase2
                idx_v[pl.ds((2 * j) * C + s, 16)] = t
                idx_v[pl.ds((2 * j + 1) * C + s, 16)] = t + 128
            return 0

        lax.fori_loop(0, NG, g_idx, 0, unroll=2)

    def fire(tab, slot):
        pltpu.async_copy(tab.at[idxs[slot]], rows[slot], sems[slot])

    def drain(tab, slot):
        pltpu.make_async_copy(tab.at[idxs[slot]], rows[slot],
                              sems[slot]).wait()

    def interp(out_ref, frow, coff, slot):
        frac_v = fracs[slot]
        rows_v = rows[slot]

        def g_interp(g, _):
            s = g * 16
            fx = frac_v[0, pl.ds(s, 16)]
            fy = frac_v[1, pl.ds(s, 16)]
            fz = frac_v[2, pl.ds(s, 16)]
            for c in range(2):
                d = [rows_v[pl.ds((2 * j + c) * C + s, 16)]
                     for j in range(8)]
                e0 = d[0] + fx * (d[1] - d[0])
                e1 = d[2] + fx * (d[3] - d[2])
                e2 = d[4] + fx * (d[5] - d[4])
                e3 = d[6] + fx * (d[7] - d[6])
                q0 = e0 + fy * (e1 - e0)
                q1 = e2 + fy * (e3 - e2)
                out_ref[frow + c, pl.ds(coff + s, 16)] = q0 + fz * (q1 - q0)
            return 0

        lax.fori_loop(0, NG, g_interp, 0, unroll=2)

    def sp_chunk(ci, _):
        pb = ci * (C * 4)

        def build_sp(l, slot):
            build(pb, l, 0, 128, 256, ST - 1, l * (ST * 2), slot)

        build_sp(0, 0)
        fire(st_hbm, 0)

        def body(k, _):
            l0 = 2 * k
            build_sp(l0 + 1, 1)
            fire(st_hbm, 1)
            drain(st_hbm, 0)
            interp(feats_sp, 2 * l0, 0, 0)
            build_sp(l0 + 2, 0)
            fire(st_hbm, 0)
            drain(st_hbm, 1)
            interp(feats_sp, 2 * l0 + 2, 0, 1)
            return 0

        lax.fori_loop(0, SL // 2 - 1, body, 0)
        build_sp(SL - 1, 1)
        fire(st_hbm, 1)
        drain(st_hbm, 0)
        interp(feats_sp, 2 * (SL - 2), 0, 0)
        drain(st_hbm, 1)
        interp(feats_sp, 2 * (SL - 1), 0, 1)
        pltpu.sync_copy(feats_sp,
                        out_hbm.at[pl.ds(0, 2 * SL),
                                   pl.ds(base_pt + ci * C, C)])
        return 0

    lax.fori_loop(0, NCHUNK, sp_chunk, 0)

    def stage(u, p):
        pltpu.async_copy(
            tt_hbm.at[pl.ds(u * (TT * F) + sid * SLICE_W, SLICE_W)],
            spms[p].at[pl.ds(sid * SLICE_W, SLICE_W)], sem_s[p])

    def wait_stage(u, p):
        pltpu.make_async_copy(
            tt_hbm.at[pl.ds(u * (TT * F) + sid * SLICE_W, SLICE_W)],
            spms[p].at[pl.ds(sid * SLICE_W, SLICE_W)], sem_s[p]).wait()

    def fire_w(u, p):
        pltpu.async_copy(flvls[p],
                         out_hbm.at[pl.ds(2 * SL + 2 * u, 2),
                                    pl.ds(base_pt, PW)], sem_w[p])

    def drain_w(u, p):
        pltpu.make_async_copy(flvls[p],
                              out_hbm.at[pl.ds(2 * SL + 2 * u, 2),
                                         pl.ds(base_pt, PW)], sem_w[p]).wait()

    def tm_level(u, p):
        wait_stage(u, p)
        plsc.subcore_barrier()
        pl.when(u + 1 < 3 * TL)(lambda: stage(u + 1, 1 - p))
        pl.when(u >= 2)(lambda: drain_w(u - 2, p))
        i = u // TL
        d0o = jnp.where(i >= 2, 128, 0)
        d1o = jnp.where(i >= 1, 256, 128)
        flvl = flvls[p]
        spm = spms[p]

        def build_tm(ci, slot):
            build(ci * (C * 4), SL + u, d0o, d1o, 384, TT - 1, 0, slot)

        build_tm(0, 0)
        fire(spm, 0)

        def body(k, _):
            c0 = 2 * k
            build_tm(c0 + 1, 1)
            fire(spm, 1)
            drain(spm, 0)
            interp(flvl, 0, c0 * C, 0)
            build_tm(c0 + 2, 0)
            fire(spm, 0)
            drain(spm, 1)
            interp(flvl, 0, (c0 + 1) * C, 1)
            return 0

        lax.fori_loop(0, NCHUNK // 2 - 1, body, 0)
        build_tm(NCHUNK - 1, 1)
        fire(spm, 1)
        drain(spm, 0)
        interp(flvl, 0, (NCHUNK - 2) * C, 0)
        drain(spm, 1)
        interp(flvl, 0, (NCHUNK - 1) * C, 1)
        fire_w(u, p)

    stage(0, 0)

    def tm_pair(k, _):
        tm_level(2 * k, 0)
        tm_level(2 * k + 1, 1)
        return 0

    lax.fori_loop(0, (3 * TL) // 2, tm_pair, 0)
    drain_w(3 * TL - 2, 0)
    drain_w(3 * TL - 1, 1)


def _encode(coords_t, st_flat, tt_flat):
    res96 = np.concatenate([RES_NP, np.tile(RES_NP[:TL], 3)])
    res_arr = jnp.asarray(np.repeat(res96[:, None], 16, axis=1))
    mesh = plsc.VectorSubcoreMesh(core_axis_name="c", subcore_axis_name="s",
                                  num_cores=NC, num_subcores=NS)
    fn = pl.kernel(
        _encode_body,
        out_type=jax.ShapeDtypeStruct((2 * (SL + 3 * TL), B), jnp.float32),
        mesh=mesh,
        scratch_types=[
            pltpu.VMEM((4 * PW,), jnp.float32),
            pltpu.VMEM((3, C), jnp.float32),
            pltpu.VMEM((3, C), jnp.float32),
            pltpu.VMEM((NRW,), jnp.int32),
            pltpu.VMEM((NRW,), jnp.int32),
            pltpu.VMEM((NRW,), jnp.float32),
            pltpu.VMEM((NRW,), jnp.float32),
            pltpu.VMEM((2 * SL, C), jnp.float32),
            pltpu.VMEM((2, PW), jnp.float32),
            pltpu.VMEM((2, PW), jnp.float32),
            pltpu.VMEM((SL + 3 * TL, 16), jnp.float32),
            pltpu.VMEM_SHARED((TT * F,), jnp.float32),
            pltpu.VMEM_SHARED((TT * F,), jnp.float32),
            pltpu.SemaphoreType.DMA,
            pltpu.SemaphoreType.DMA,
            pltpu.SemaphoreType.DMA,
            pltpu.SemaphoreType.DMA,
            pltpu.SemaphoreType.DMA,
            pltpu.SemaphoreType.DMA,
        ],
    )
    return fn(coords_t, st_flat, tt_flat, res_arr)


def _mlp_body(x_ref, w1_ref, b1_ref, g1_ref, be1_ref,
              w2_ref, b2_ref, g2_ref, be2_ref, w3_ref, b3_ref, o_ref):
    x = x_ref[...]
    h = lax.dot_general(w1_ref[...], x, (((0,), (0,)), ((), ())),
                        preferred_element_type=jnp.float32,
                        precision=lax.Precision.HIGHEST)
    h = jnp.maximum(h + b1_ref[...], 0.0)
    m = jnp.mean(h, axis=0, keepdims=True)
    v = jnp.mean((h - m) ** 2, axis=0, keepdims=True)
    h = (h - m) * lax.rsqrt(v + 1e-5) * g1_ref[...] + be1_ref[...]
    h = lax.dot_general(w2_ref[...], h, (((0,), (0,)), ((), ())),
                        preferred_element_type=jnp.float32,
                        precision=lax.Precision.HIGHEST)
    h = jnp.maximum(h + b2_ref[...], 0.0)
    m = jnp.mean(h, axis=0, keepdims=True)
    v = jnp.mean((h - m) ** 2, axis=0, keepdims=True)
    h = (h - m) * lax.rsqrt(v + 1e-5) * g2_ref[...] + be2_ref[...]
    o = lax.dot_general(w3_ref[...], h, (((0,), (0,)), ((), ())),
                        preferred_element_type=jnp.float32,
                        precision=lax.Precision.HIGHEST)
    o_ref[...] = o + b3_ref[...]


def _mlp(feats, W1, b1, g1, be1, W2, b2, g2, be2, W3, b3):
    D = feats.shape[0]
    H = W1.shape[1]
    BT = 2048
    grid = (B // BT,)
    full = lambda i: (0, 0)
    out = pl.pallas_call(
        _mlp_body,
        grid=grid,
        in_specs=[
            pl.BlockSpec((D, BT), lambda i: (0, i)),
            pl.BlockSpec((D, H), full),
            pl.BlockSpec((H, 1), full),
            pl.BlockSpec((H, 1), full),
            pl.BlockSpec((H, 1), full),
            pl.BlockSpec((H, H), full),
            pl.BlockSpec((H, 1), full),
            pl.BlockSpec((H, 1), full),
            pl.BlockSpec((H, 1), full),
            pl.BlockSpec((H, 1), full),
            pl.BlockSpec((1, 1), full),
        ],
        out_specs=pl.BlockSpec((1, BT), lambda i: (0, i)),
        out_shape=jax.ShapeDtypeStruct((1, B), jnp.float32),
    )(feats, W1, b1.reshape(H, 1), g1.reshape(H, 1), be1.reshape(H, 1),
      W2, b2.reshape(H, 1), g2.reshape(H, 1), be2.reshape(H, 1),
      W3, b3.reshape(1, 1))
    return out[0]


def kernel(coords, spatial_table, temporal_table,
           W1, b1, g1, be1, W2, b2, g2, be2, W3, b3):
    st_flat = (spatial_table.reshape(SL, ST // 128, 128, F)
               .transpose(0, 1, 3, 2).reshape(SL * ST * F))
    tt_flat = (temporal_table.reshape(3, TL, TT // 128, 128, F)
               .transpose(0, 1, 2, 4, 3).reshape(3 * TL * TT * F))
    coords_f = (coords.reshape(B // 128, 128, 4)
                .transpose(0, 2, 1).reshape(B * 4))
    feats = _encode(coords_f, st_flat, tt_flat)
    return _mlp(feats, W1, b1, g1, be1, W2, b2, g2, be2, W3, b3)


# speedup vs baseline: 213.0931x; 1.6515x over previous
import functools

import numpy as np
import jax
import jax.numpy as jnp
from jax import lax
from jax.experimental import pallas as pl
from jax.experimental.pallas import tpu as pltpu
from jax.experimental.pallas import tpu_sc as plsc

SL = 36
TL = 20
ST = 2 ** 19
TT = 2 ** 16
F = 2
B = 131072
BASE = 16.0
GROWTH = 1.3819

P1_I32 = np.int32(np.asarray(2654435761, np.uint32).view(np.int32))
P2_I32 = np.int32(805459861)

RES_NP = np.array([np.floor(BASE * GROWTH ** l) for l in range(SL)], np.float32)

NC = 2
NS = 16
NW = NC * NS
PW = B // NW
C = 256
NCHUNK = PW // C
NG = C // 16
NR = C * 8

_SECTIONS = (
    ((0, 1, 2), SL, ST - 1, 0),
    ((0, 1, 3), TL, TT - 1, 2 * SL),
    ((0, 2, 3), TL, TT - 1, 2 * SL + 2 * TL),
    ((1, 2, 3), TL, TT - 1, 2 * SL + 4 * TL),
)


NRW = NR * F
SLICE_W = (TT * F) // NS


def _encode_body(coords_hbm, st_hbm, tt_hbm, res_hbm, out_hbm,
                 coords_v, frac_a, frac_b, idx_a, idx_b, rows_a, rows_b,
                 flvl_a, flvl_b, res_v, spm_sp,
                 sem_a, sem_b, sem_wa, sem_wb, sem_ss, sem_st):
    cid = lax.axis_index("c")
    sid = lax.axis_index("s")
    wid = sid * NC + cid
    base_pt = wid * PW
    pltpu.sync_copy(res_hbm, res_v)
    pltpu.sync_copy(coords_hbm.at[pl.ds(base_pt * 4, PW * 4)], coords_v)
    sems = (sem_a, sem_b)
    fracs = (frac_a, frac_b)
    idxs = (idx_a, idx_b)
    rows = (rows_a, rows_b)
    sem_w = (sem_wa, sem_wb)
    flvls = (flvl_a, flvl_b)

    def build(pb, resrow, d0o, d1o, d2o, mask, tbase2, slot):
        resv = res_v[resrow, :]
        frac_v = fracs[slot]
        idx_v = idxs[slot]

        def g_idx(g, _):
            s = g * 16
            cb = pb + (g >> 3) * 512 + (g & 7) * 16
            x = coords_v[pl.ds(cb + d0o, 16)] * resv
            y = coords_v[pl.ds(cb + d1o, 16)] * resv
            z = coords_v[pl.ds(cb + d2o, 16)] * resv
            xi = x.astype(jnp.int32)
            yi = y.astype(jnp.int32)
            zi = z.astype(jnp.int32)
            frac_v[0, pl.ds(s, 16)] = x - xi.astype(jnp.float32)
            frac_v[1, pl.ds(s, 16)] = y - yi.astype(jnp.float32)
            frac_v[2, pl.ds(s, 16)] = z - zi.astype(jnp.float32)
            hx0 = xi
            hx1 = xi + 1
            hy0 = yi * P1_I32
            hy1 = hy0 + P1_I32
            hz0 = zi * P2_I32
            hz1 = hz0 + P2_I32
            for j in range(8):
                h = ((hx1 if j & 1 else hx0)
                     ^ (hy1 if j & 2 else hy0)
                     ^ (hz1 if j & 4 else hz0))
                r = h & mask
                t = (r + (r & -128)) + tbase2
                idx_v[pl.ds((2 * j) * C + s, 16)] = t
                idx_v[pl.ds((2 * j + 1) * C + s, 16)] = t + 128
            return 0

        lax.fori_loop(0, NG, g_idx, 0, unroll=2)

    def fire(tab, slot):
        pltpu.async_copy(tab.at[idxs[slot]], rows[slot], sems[slot])

    def drain(tab, slot):
        pltpu.make_async_copy(tab.at[idxs[slot]], rows[slot],
                              sems[slot]).wait()

    def interp(out_ref, frow, coff, slot):
        frac_v = fracs[slot]
        rows_v = rows[slot]

        def g_interp(g, _):
            s = g * 16
            fx = frac_v[0, pl.ds(s, 16)]
            fy = frac_v[1, pl.ds(s, 16)]
            fz = frac_v[2, pl.ds(s, 16)]
            for c in range(2):
                d = [rows_v[pl.ds((2 * j + c) * C + s, 16)]
                     for j in range(8)]
                e0 = d[0] + fx * (d[1] - d[0])
                e1 = d[2] + fx * (d[3] - d[2])
                e2 = d[4] + fx * (d[5] - d[4])
                e3 = d[6] + fx * (d[7] - d[6])
                q0 = e0 + fy * (e1 - e0)
                q1 = e2 + fy * (e3 - e2)
                out_ref[frow + c, pl.ds(coff + s, 16)] = q0 + fz * (q1 - q0)
            return 0

        lax.fori_loop(0, NG, g_interp, 0, unroll=2)

    SLICE_SP = (ST * F) // NS

    def stage_sp(l):
        pltpu.async_copy(
            st_hbm.at[pl.ds(l * (ST * F) + sid * SLICE_SP, SLICE_SP)],
            spm_sp.at[pl.ds(sid * SLICE_SP, SLICE_SP)], sem_ss)

    def wait_stage_sp(l):
        pltpu.make_async_copy(
            st_hbm.at[pl.ds(l * (ST * F) + sid * SLICE_SP, SLICE_SP)],
            spm_sp.at[pl.ds(sid * SLICE_SP, SLICE_SP)], sem_ss).wait()

    def fire_w(frow, p):
        pltpu.async_copy(flvls[p],
                         out_hbm.at[pl.ds(frow, 2),
                                    pl.ds(base_pt, PW)], sem_w[p])

    def drain_w(frow, p):
        pltpu.make_async_copy(flvls[p],
                              out_hbm.at[pl.ds(frow, 2),
                                         pl.ds(base_pt, PW)], sem_w[p]).wait()

    def level_chunks(spm, build_l, flvl):
        build_l(0, 0)
        fire(spm, 0)

        def body(k, _):
            c0 = 2 * k
            build_l(c0 + 1, 1)
            fire(spm, 1)
            drain(spm, 0)
            interp(flvl, 0, c0 * C, 0)
            build_l(c0 + 2, 0)
            fire(spm, 0)
            drain(spm, 1)
            interp(flvl, 0, (c0 + 1) * C, 1)
            return 0

        lax.fori_loop(0, NCHUNK // 2 - 1, body, 0)
        build_l(NCHUNK - 1, 1)
        fire(spm, 1)
        drain(spm, 0)
        interp(flvl, 0, (NCHUNK - 2) * C, 0)
        drain(spm, 1)
        interp(flvl, 0, (NCHUNK - 1) * C, 1)

    def sp_pair(k, _):
        for p in range(2):
            l = 2 * k + p
            plsc.subcore_barrier()
            stage_sp(l)
            wait_stage_sp(l)
            plsc.subcore_barrier()
            pl.when(l >= 2)(lambda: drain_w(2 * (l - 2), p))

            def build_sp(ci, slot, l=l):
                build(ci * (C * 4), l, 0, 128, 256, ST - 1, 0, slot)

            level_chunks(spm_sp, build_sp, flvls[p])
            fire_w(2 * l, p)
        return 0

    lax.fori_loop(0, SL // 2, sp_pair, 0)
    drain_w(2 * (SL - 2), 0)
    drain_w(2 * (SL - 1), 1)

    def stage_tm(u):
        pltpu.async_copy(
            tt_hbm.at[pl.ds(u * (TT * F) + sid * SLICE_W, SLICE_W)],
            spm_sp.at[pl.ds(sid * SLICE_W, SLICE_W)], sem_st)

    def wait_stage_tm(u):
        pltpu.make_async_copy(
            tt_hbm.at[pl.ds(u * (TT * F) + sid * SLICE_W, SLICE_W)],
            spm_sp.at[pl.ds(sid * SLICE_W, SLICE_W)], sem_st).wait()

    def tm_pair(k, _):
        for p in range(2):
            u = 2 * k + p
            plsc.subcore_barrier()
            stage_tm(u)
            wait_stage_tm(u)
            plsc.subcore_barrier()
            pl.when(u >= 2)(lambda: drain_w(2 * SL + 2 * (u - 2), p))
            i = u // TL
            d0o = jnp.where(i >= 2, 128, 0)
            d1o = jnp.where(i >= 1, 256, 128)

            def build_tm(ci, slot, u=u, d0o=d0o, d1o=d1o):
                build(ci * (C * 4), SL + u, d0o, d1o, 384, TT - 1, 0, slot)

            level_chunks(spm_sp, build_tm, flvls[p])
            fire_w(2 * SL + 2 * u, p)
        return 0

    lax.fori_loop(0, (3 * TL) // 2, tm_pair, 0)
    drain_w(2 * SL + 2 * (3 * TL - 2), 0)
    drain_w(2 * SL + 2 * (3 * TL - 1), 1)


def _encode(coords_t, st_flat, tt_flat):
    res96 = np.concatenate([RES_NP, np.tile(RES_NP[:TL], 3)])
    res_arr = jnp.asarray(np.repeat(res96[:, None], 16, axis=1))
    mesh = plsc.VectorSubcoreMesh(core_axis_name="c", subcore_axis_name="s",
                                  num_cores=NC, num_subcores=NS)
    fn = pl.kernel(
        _encode_body,
        out_type=jax.ShapeDtypeStruct((2 * (SL + 3 * TL), B), jnp.float32),
        mesh=mesh,
        scratch_types=[
            pltpu.VMEM((4 * PW,), jnp.float32),
            pltpu.VMEM((3, C), jnp.float32),
            pltpu.VMEM((3, C), jnp.float32),
            pltpu.VMEM((NRW,), jnp.int32),
            pltpu.VMEM((NRW,), jnp.int32),
            pltpu.VMEM((NRW,), jnp.float32),
            pltpu.VMEM((NRW,), jnp.float32),
            pltpu.VMEM((2, PW), jnp.float32),
            pltpu.VMEM((2, PW), jnp.float32),
            pltpu.VMEM((SL + 3 * TL, 16), jnp.float32),
            pltpu.VMEM_SHARED((ST * F,), jnp.float32),
            pltpu.SemaphoreType.DMA,
            pltpu.SemaphoreType.DMA,
            pltpu.SemaphoreType.DMA,
            pltpu.SemaphoreType.DMA,
            pltpu.SemaphoreType.DMA,
            pltpu.SemaphoreType.DMA,
        ],
    )
    return fn(coords_t, st_flat, tt_flat, res_arr)


def _mlp_body(x_ref, w1_ref, b1_ref, g1_ref, be1_ref,
              w2_ref, b2_ref, g2_ref, be2_ref, w3_ref, b3_ref, o_ref):
    x = x_ref[...]
    h = lax.dot_general(w1_ref[...], x, (((0,), (0,)), ((), ())),
                        preferred_element_type=jnp.float32,
                        precision=lax.Precision.HIGHEST)
    h = jnp.maximum(h + b1_ref[...], 0.0)
    m = jnp.mean(h, axis=0, keepdims=True)
    v = jnp.mean((h - m) ** 2, axis=0, keepdims=True)
    h = (h - m) * lax.rsqrt(v + 1e-5) * g1_ref[...] + be1_ref[...]
    h = lax.dot_general(w2_ref[...], h, (((0,), (0,)), ((), ())),
                        preferred_element_type=jnp.float32,
                        precision=lax.Precision.HIGHEST)
    h = jnp.maximum(h + b2_ref[...], 0.0)
    m = jnp.mean(h, axis=0, keepdims=True)
    v = jnp.mean((h - m) ** 2, axis=0, keepdims=True)
    h = (h - m) * lax.rsqrt(v + 1e-5) * g2_ref[...] + be2_ref[...]
    o = lax.dot_general(w3_ref[...], h, (((0,), (0,)), ((), ())),
                        preferred_element_type=jnp.float32,
                        precision=lax.Precision.HIGHEST)
    o_ref[...] = o + b3_ref[...]


def _mlp(feats, W1, b1, g1, be1, W2, b2, g2, be2, W3, b3):
    D = feats.shape[0]
    H = W1.shape[1]
    BT = 2048
    grid = (B // BT,)
    full = lambda i: (0, 0)
    out = pl.pallas_call(
        _mlp_body,
        grid=grid,
        in_specs=[
            pl.BlockSpec((D, BT), lambda i: (0, i)),
            pl.BlockSpec((D, H), full),
            pl.BlockSpec((H, 1), full),
            pl.BlockSpec((H, 1), full),
            pl.BlockSpec((H, 1), full),
            pl.BlockSpec((H, H), full),
            pl.BlockSpec((H, 1), full),
            pl.BlockSpec((H, 1), full),
            pl.BlockSpec((H, 1), full),
            pl.BlockSpec((H, 1), full),
            pl.BlockSpec((1, 1), full),
        ],
        out_specs=pl.BlockSpec((1, BT), lambda i: (0, i)),
        out_shape=jax.ShapeDtypeStruct((1, B), jnp.float32),
    )(feats, W1, b1.reshape(H, 1), g1.reshape(H, 1), be1.reshape(H, 1),
      W2, b2.reshape(H, 1), g2.reshape(H, 1), be2.reshape(H, 1),
      W3, b3.reshape(1, 1))
    return out[0]


def kernel(coords, spatial_table, temporal_table,
           W1, b1, g1, be1, W2, b2, g2, be2, W3, b3):
    st_flat = (spatial_table.reshape(SL, ST // 128, 128, F)
               .transpose(0, 1, 3, 2).reshape(SL * ST * F))
    tt_flat = (temporal_table.reshape(3, TL, TT // 128, 128, F)
               .transpose(0, 1, 2, 4, 3).reshape(3 * TL * TT * F))
    coords_f = (coords.reshape(B // 128, 128, 4)
                .transpose(0, 2, 1).reshape(B * 4))
    feats = _encode(coords_f, st_flat, tt_flat)
    return _mlp(feats, W1, b1, g1, be1, W2, b2, g2, be2, W3, b3)
